# baseline (speedup 1.0000x reference)
import functools

import jax
import jax.numpy as jnp
from jax import lax
from jax.experimental import pallas as pl
from jax.experimental.pallas import tpu as pltpu
from jax.experimental.pallas import tpu_sc as plsc

NUM_WORKERS = 32
CHUNK = 128
K = 4
SUP = K * CHUNK


@functools.partial(jax.jit, static_argnums=(2, 3, 4, 5))
def _gather_tr(idxt, table, total, d, bsz, h):
    b_per_w = total // NUM_WORKERS
    n_sup = b_per_w // SUP
    nt4 = h * (d // 8)
    rows_out = bsz // 128 * 8

    mesh = plsc.VectorSubcoreMesh(core_axis_name="c", subcore_axis_name="s")

    @functools.partial(
        pl.kernel,
        mesh=mesh,
        out_type=jax.ShapeDtypeStruct((nt4, rows_out, 128), jnp.float32),
        scratch_types=[
            pltpu.VMEM((2, K, CHUNK), jnp.int32),
            pltpu.VMEM((2, SUP, 32), jnp.float32),
            pltpu.VMEM((2, K * 32, 128), jnp.float32),
            pltpu.SemaphoreType.DMA,
            pltpu.SemaphoreType.DMA,
            pltpu.SemaphoreType.DMA,
            pltpu.SemaphoreType.DMA,
            pltpu.SemaphoreType.DMA,
            pltpu.SemaphoreType.DMA,
        ],
        compiler_params=pltpu.CompilerParams(use_tc_tiling_on_sc=False,
                                             needs_layout_passes=False),
    )
    def k(idx_hbm, table_hbm, out_hbm, idx_v, rows_v, ov_v,
          i_sem0, i_sem1, g_sem0, g_sem1, o_sem0, o_sem1):
        wid = lax.axis_index("s") * 2 + lax.axis_index("c")
        i_sems = (i_sem0, i_sem1)
        g_sems = (g_sem0, g_sem1)
        o_sems = (o_sem0, o_sem1)
        iot = lax.iota(jnp.int32, 16)

        def prefetch_idx(s, p):
            blk = wid * n_sup + jnp.minimum(s, n_sup - 1)
            pltpu.async_copy(idx_hbm.at[blk], idx_v.at[p], i_sems[p])

        def wait_idx(p):
            pltpu.make_async_copy(idx_hbm.at[0], idx_v.at[p], i_sems[p]).wait()

        def fire_gathers(p):
            for j in range(K):
                pltpu.async_copy(table_hbm.at[idx_v.at[p, j]],
                                 rows_v.at[p, pl.ds(j * CHUNK, CHUNK)],
                                 g_sems[p])

        def drain_gathers(p):
            for j in range(K):
                pltpu.make_async_copy(table_hbm.at[pl.ds(0, CHUNK)],
                                      rows_v.at[p, pl.ds(j * CHUNK, CHUNK)],
                                      g_sems[p]).wait()

        def drain_out(p):
            for dt in range(4):
                pltpu.make_async_copy(out_hbm.at[0, pl.ds(0, K * 8)],
                                      ov_v.at[p, pl.ds(dt * K * 8, K * 8)],
                                      o_sems[p]).wait()

        def transpose_block(p):
            rows = rows_v.at[p]

            def tr_body(i2, carry):
                c = i2 // 32
                dd = i2 % 32
                col = jnp.full((16,), dd, jnp.int32)
                r = (dd // 8) * (K * 8) + c * 8 + (dd % 8)
                for g3 in range(8):
                    ridx = c * CHUNK + g3 * 16 + iot
                    vec = plsc.load_gather(rows, [ridx, col])
                    ov_v[p, r, pl.ds(g3 * 16, 16)] = vec
                return carry

            lax.fori_loop(0, K * 32, tr_body, 0)

        def fire_out(s, p):
            c0 = wid * n_sup * K + s * K
            t = c0 // 128
            r0 = (c0 % 128) * 8
            for dt in range(4):
                pltpu.async_copy(ov_v.at[p, pl.ds(dt * K * 8, K * 8)],
                                 out_hbm.at[t * 4 + dt, pl.ds(r0, K * 8)],
                                 o_sems[p])

        def do_block(s, p, first):
            q = 1 - p
            drain_gathers(p)
            wait_idx(q)
            fire_gathers(q)
            prefetch_idx(s + 2, p)
            if not first:
                drain_out(p)
            transpose_block(p)
            fire_out(s, p)

        pltpu.sync_copy(idx_hbm.at[wid * n_sup], idx_v.at[0])
        fire_gathers(0)
        prefetch_idx(1, 1)
        do_block(0, 0, first=True)
        do_block(1, 1, first=True)

        def body(g, carry):
            do_block(2 * g, 0, first=False)
            do_block(2 * g + 1, 1, first=False)
            return carry

        lax.fori_loop(1, n_sup // 2, body, 0)

        drain_gathers(0)
        pltpu.make_async_copy(idx_hbm.at[0], idx_v.at[1], i_sems[1]).wait()
        drain_out(0)
        drain_out(1)

    return k(idxt.reshape(total // SUP, K, CHUNK), table)


def kernel(indices, table):
    bsz, h = indices.shape
    v, d = table.shape
    total = bsz * h
    idxt = indices.T.reshape(total).astype(jnp.int32)
    out3 = _gather_tr(idxt, table, total, d, bsz, h)
    out5 = out3.reshape(h, d // 8, bsz // 128, 8, 128)
    return out5.transpose(2, 4, 0, 1, 3).reshape(bsz, h, d)


# speedup vs baseline: 1.6433x; 1.6433x over previous
import functools

import jax
import jax.numpy as jnp
from jax import lax
from jax.experimental import pallas as pl
from jax.experimental.pallas import tpu as pltpu
from jax.experimental.pallas import tpu_sc as plsc

NUM_WORKERS = 32
CHUNK = 128
K = 4
SUP = K * CHUNK


@functools.partial(jax.jit, static_argnums=(2, 3, 4, 5))
def _gather_tr(idxt, table, total, d, bsz, h):
    b_per_w = total // NUM_WORKERS
    n_sup = b_per_w // SUP
    nt4 = h * (d // 8)
    rows_out = bsz // 128 * 8

    mesh = plsc.VectorSubcoreMesh(core_axis_name="c", subcore_axis_name="s")

    @functools.partial(
        pl.kernel,
        mesh=mesh,
        out_type=jax.ShapeDtypeStruct((nt4, rows_out, 128), jnp.float32),
        scratch_types=[
            pltpu.VMEM((2, K, CHUNK), jnp.int32),
            pltpu.VMEM((2, SUP, 32), jnp.float32),
            pltpu.VMEM((2, K * 32, 128), jnp.float32),
            pltpu.SemaphoreType.DMA,
            pltpu.SemaphoreType.DMA,
            pltpu.SemaphoreType.DMA,
            pltpu.SemaphoreType.DMA,
            pltpu.SemaphoreType.DMA,
            pltpu.SemaphoreType.DMA,
        ],
        compiler_params=pltpu.CompilerParams(use_tc_tiling_on_sc=False,
                                             needs_layout_passes=False),
    )
    def k(idx_hbm, table_hbm, out_hbm, idx_v, rows_v, ov_v,
          i_sem0, i_sem1, g_sem0, g_sem1, o_sem0, o_sem1):
        wid = lax.axis_index("s") * 2 + lax.axis_index("c")
        i_sems = (i_sem0, i_sem1)
        g_sems = (g_sem0, g_sem1)
        o_sems = (o_sem0, o_sem1)
        iot = lax.iota(jnp.int32, 16)

        def prefetch_idx(s, p):
            blk = wid * n_sup + jnp.minimum(s, n_sup - 1)
            pltpu.async_copy(idx_hbm.at[blk], idx_v.at[p], i_sems[p])

        def wait_idx(p):
            pltpu.make_async_copy(idx_hbm.at[0], idx_v.at[p], i_sems[p]).wait()

        def fire_gathers(p):
            for j in range(K):
                pltpu.async_copy(table_hbm.at[idx_v.at[p, j]],
                                 rows_v.at[p, pl.ds(j * CHUNK, CHUNK)],
                                 g_sems[p])

        def drain_gathers(p):
            for j in range(K):
                pltpu.make_async_copy(table_hbm.at[pl.ds(0, CHUNK)],
                                      rows_v.at[p, pl.ds(j * CHUNK, CHUNK)],
                                      g_sems[p]).wait()

        def drain_out(p):
            for dt in range(4):
                pltpu.make_async_copy(out_hbm.at[0, pl.ds(0, K * 8)],
                                      ov_v.at[p, pl.ds(dt * K * 8, K * 8)],
                                      o_sems[p]).wait()

        def transpose_block(p):
            rows = rows_v.at[p]

            @plsc.parallel_loop(0, K * 32, unroll=4)
            def tr_body(i2):
                c = i2 // 32
                dd = i2 % 32
                col = jnp.full((16,), dd, jnp.int32)
                r = (dd // 8) * (K * 8) + c * 8 + (dd % 8)
                for g3 in range(8):
                    ridx = c * CHUNK + g3 * 16 + iot
                    vec = plsc.load_gather(rows, [ridx, col])
                    ov_v[p, r, pl.ds(g3 * 16, 16)] = vec

        def fire_out(s, p):
            c0 = wid * n_sup * K + s * K
            t = c0 // 128
            r0 = (c0 % 128) * 8
            for dt in range(4):
                pltpu.async_copy(ov_v.at[p, pl.ds(dt * K * 8, K * 8)],
                                 out_hbm.at[t * 4 + dt, pl.ds(r0, K * 8)],
                                 o_sems[p])

        def do_block(s, p, first):
            q = 1 - p
            drain_gathers(p)
            wait_idx(q)
            fire_gathers(q)
            prefetch_idx(s + 2, p)
            if not first:
                drain_out(p)
            transpose_block(p)
            fire_out(s, p)

        pltpu.sync_copy(idx_hbm.at[wid * n_sup], idx_v.at[0])
        fire_gathers(0)
        prefetch_idx(1, 1)
        do_block(0, 0, first=True)
        do_block(1, 1, first=True)

        def body(g, carry):
            do_block(2 * g, 0, first=False)
            do_block(2 * g + 1, 1, first=False)
            return carry

        lax.fori_loop(1, n_sup // 2, body, 0)

        drain_gathers(0)
        pltpu.make_async_copy(idx_hbm.at[0], idx_v.at[1], i_sems[1]).wait()
        drain_out(0)
        drain_out(1)

    return k(idxt.reshape(total // SUP, K, CHUNK), table)


def kernel(indices, table):
    bsz, h = indices.shape
    v, d = table.shape
    total = bsz * h
    idxt = indices.T.reshape(total).astype(jnp.int32)
    out3 = _gather_tr(idxt, table, total, d, bsz, h)
    out5 = out3.reshape(h, d // 8, bsz // 128, 8, 128)
    return out5.transpose(2, 4, 0, 1, 3).reshape(bsz, h, d)
